# baseline (speedup 1.0000x reference)
import jax
import jax.numpy as jnp
from jax import lax
from jax.experimental import pallas as pl
from jax.experimental.pallas import tpu as pltpu
from jax.experimental.pallas import tpu_sc as plsc

B = 16384
F = 26
D = 4
W = 8
NC, NS, L = 2, 16, 16
NW = NC * NS
EPW = B // NW
IPW = EPW * F
CH = 4
ECH = EPW // CH
ICH = ECH * F
GCH = ECH // L


def _fm_kernel(ids_hbm, bias_hbm, tab_hbm,
               logits_hbm, adj_hbm, prob_hbm,
               idx_v, rows_v0, rows_v1, bias_v, logit_v, prob_v,
               sem0, sem1):
    wid = lax.axis_index("s") * NC + lax.axis_index("c")
    id_base = wid * IPW
    ex_base = wid * EPW

    bufs = [(rows_v0, sem0), (rows_v1, sem1)]

    def fire(c):
        rows_v, sem = bufs[c % 2]
        pltpu.sync_copy(ids_hbm.at[pl.ds(id_base + c * ICH, ICH)],
                        idx_v.at[c])
        return pltpu.async_copy(tab_hbm.at[idx_v.at[c]], rows_v, sem)

    pending = [fire(0), fire(1)]
    pltpu.sync_copy(bias_hbm, bias_v)

    iota = lax.iota(jnp.int32, L)
    row_base = iota * F
    d_idx = [jnp.full((L,), d, jnp.int32) for d in range(D + 1)]
    bias_vec = bias_v[...]
    zero_f = jnp.zeros((L,), jnp.float32)

    for c in range(CH):
        rows_v, _ = bufs[c % 2]
        pending[c % 2].wait()

        def group_body(g, carry):
            r0 = row_base + g * (L * F)
            acc = [zero_f] * D
            accsq = [zero_f] * D
            lin = zero_f
            for f in range(F):
                r = r0 + f
                for d in range(D):
                    v = plsc.load_gather(rows_v, [r, d_idx[d]])
                    acc[d] = acc[d] + v
                    accsq[d] = accsq[d] + v * v
                lin = lin + plsc.load_gather(rows_v, [r, d_idx[D]])
            cross = zero_f
            for d in range(D):
                cross = cross + (acc[d] * acc[d] - accsq[d])
            logits = bias_vec + lin + 0.5 * cross
            prob = 1.0 / (1.0 + jnp.exp(-logits))
            logit_v[pl.ds(c * ECH + g * L, L)] = logits
            prob_v[pl.ds(c * ECH + g * L, L)] = prob
            return carry

        lax.fori_loop(0, GCH, group_body, 0)

        if c + 2 < CH:
            pending[c % 2] = fire(c + 2)

    pltpu.sync_copy(logit_v, logits_hbm.at[pl.ds(ex_base, EPW)])
    pltpu.sync_copy(logit_v, adj_hbm.at[pl.ds(ex_base, EPW)])
    pltpu.sync_copy(prob_v, prob_hbm.at[pl.ds(ex_base, EPW)])


@jax.jit
def kernel(feature_ids, linear_bias, linear_w, cross_emb_w):
    ids_flat = feature_ids.reshape(-1)
    bias16 = jnp.broadcast_to(linear_bias, (L,)).astype(jnp.float32)
    table = jnp.concatenate(
        [cross_emb_w, linear_w,
         jnp.zeros((cross_emb_w.shape[0], W - D - 1), jnp.float32)], axis=1)

    run = pl.kernel(
        _fm_kernel,
        out_type=(
            jax.ShapeDtypeStruct((B,), jnp.float32),
            jax.ShapeDtypeStruct((B,), jnp.float32),
            jax.ShapeDtypeStruct((B,), jnp.float32),
        ),
        mesh=plsc.VectorSubcoreMesh(core_axis_name="c", subcore_axis_name="s"),
        compiler_params=pltpu.CompilerParams(
            needs_layout_passes=False, use_tc_tiling_on_sc=False),
        scratch_types=[
            pltpu.VMEM((CH, ICH), jnp.int32),
            pltpu.VMEM((ICH, W), jnp.float32),
            pltpu.VMEM((ICH, W), jnp.float32),
            pltpu.VMEM((L,), jnp.float32),
            pltpu.VMEM((EPW,), jnp.float32),
            pltpu.VMEM((EPW,), jnp.float32),
            pltpu.SemaphoreType.DMA,
            pltpu.SemaphoreType.DMA,
        ],
    )
    logits, adj, prob = run(ids_flat, bias16, table)
    return (logits[:, None], adj[:, None], prob[:, None])
